# baseline (speedup 1.0000x reference)
import functools

import jax
import jax.numpy as jnp
from jax import lax
from jax.experimental import pallas as pl
from jax.experimental.pallas import tpu as pltpu
from jax.experimental.pallas import tpu_sc as plsc

N_TOKENS = 8192
N_CODES = 8192
D = 32
TB = 128

NC = 2
NS = 16
NW = NC * NS
BPW = N_TOKENS // NW
HALF = BPW // 2


def _argmin_body(x_ref, ehi_ref, elo_ref, x2_ref, e2_ref, idx_ref):
    x = x_ref[...]
    e_hi = ehi_ref[...]
    e_lo = elo_ref[...]
    x2 = x2_ref[...]
    e2 = e2_ref[...]
    lhs = (2.0 * x).astype(jnp.bfloat16)
    dn = (((1,), (1,)), ((), ()))
    p_hi = lax.dot_general(lhs, e_hi, dn, preferred_element_type=jnp.float32)
    p_lo = lax.dot_general(lhs, e_lo, dn, preferred_element_type=jnp.float32)
    dist = (x2 - (p_hi + p_lo)) + e2
    m = jnp.min(dist, axis=1, keepdims=True)
    cols = lax.broadcasted_iota(jnp.int32, dist.shape, 1)
    idx = jnp.min(jnp.where(dist == m, cols, jnp.int32(2**30)), axis=1)
    idx_ref[0, 0, :] = idx


def _stage1(x, e):
    x2 = (x ** 2).sum(axis=1, keepdims=True)
    e2 = (e ** 2).sum(axis=1)[None, :]
    e_hi = e.astype(jnp.bfloat16)
    e_lo = (e - e_hi.astype(jnp.float32)).astype(jnp.bfloat16)
    grid = N_TOKENS // TB
    out = pl.pallas_call(
        _argmin_body,
        grid=(grid,),
        in_specs=[
            pl.BlockSpec((TB, D), lambda i: (i, 0)),
            pl.BlockSpec((N_CODES, D), lambda i: (0, 0)),
            pl.BlockSpec((N_CODES, D), lambda i: (0, 0)),
            pl.BlockSpec((TB, 1), lambda i: (i, 0)),
            pl.BlockSpec((1, N_CODES), lambda i: (0, 0)),
        ],
        out_specs=pl.BlockSpec((1, 1, TB), lambda i: (i, 0, 0)),
        out_shape=jax.ShapeDtypeStruct((grid, 1, TB), jnp.int32),
    )(x, e_hi, e_lo, x2, e2)
    return out.reshape(N_TOKENS)


def _sc_body(e_hbm, idx_hbm, emb_out, cnt_out,
             idx_a, idx_b, rows_a, rows_b, cnt_v, sem):
    cid = lax.axis_index("c")
    sid = lax.axis_index("s")
    wid = sid * NC + cid
    base = wid * BPW

    pltpu.sync_copy(idx_hbm.at[pl.ds(base, HALF)], idx_a)
    pltpu.sync_copy(idx_hbm.at[pl.ds(base + HALF, HALF)], idx_b)

    zeros16 = jnp.zeros((16,), jnp.int32)

    def _zero(i, carry):
        cnt_v[pl.ds(i * 16, 16)] = zeros16
        return carry

    lax.fori_loop(0, N_CODES // 16, _zero, 0)

    pltpu.async_copy(e_hbm.at[idx_a], rows_a, sem).wait()
    pltpu.async_copy(e_hbm.at[idx_b], rows_b, sem).wait()
    pltpu.sync_copy(rows_a, emb_out.at[pl.ds(base, HALF)])
    pltpu.sync_copy(rows_b, emb_out.at[pl.ds(base + HALF, HALF)])

    ones16 = jnp.ones((16,), jnp.int32)

    def _hist(ref):
        def body(i, carry):
            v = ref[pl.ds(i * 16, 16)]
            plsc.addupdate_scatter(cnt_v, [v], ones16)
            return carry
        lax.fori_loop(0, HALF // 16, body, 0)

    _hist(idx_a)
    _hist(idx_b)

    pltpu.sync_copy(cnt_v, cnt_out.at[wid])


@functools.cache
def _make_sc_gather():
    return pl.kernel(
        _sc_body,
        out_type=(
            jax.ShapeDtypeStruct((N_TOKENS, D), jnp.float32),
            jax.ShapeDtypeStruct((NW, N_CODES), jnp.int32),
        ),
        mesh=plsc.VectorSubcoreMesh(
            core_axis_name="c", subcore_axis_name="s",
            num_cores=NC, num_subcores=NS,
        ),
        scratch_types=[
            pltpu.VMEM((HALF,), jnp.int32),
            pltpu.VMEM((HALF,), jnp.int32),
            pltpu.VMEM((HALF, D), jnp.float32),
            pltpu.VMEM((HALF, D), jnp.float32),
            pltpu.VMEM((N_CODES,), jnp.int32),
            pltpu.SemaphoreType.DMA,
        ],
        compiler_params=pltpu.CompilerParams(
            needs_layout_passes=False, use_tc_tiling_on_sc=False),
    )


def _final_body(x_ref, emb_ref, cnt_ref, st_ref, loss_ref, perp_ref,
                ncode_ref, ratio_ref):
    x = x_ref[...]
    emb = emb_ref[...]
    d = emb - x
    st_ref[...] = d + x
    mse = jnp.mean(d * d)
    loss_ref[...] = jnp.reshape(jnp.minimum(mse, 10.0) * 0.25, (1, 1))

    cnt2 = cnt_ref[...]
    cnt = jnp.sum(cnt2, axis=0, keepdims=True)
    avg = cnt.astype(jnp.float32) * (1.0 / N_TOKENS)
    ent = jnp.sum(avg * jnp.log(avg + 1e-7))
    perp_ref[...] = jnp.reshape(jnp.exp(-ent), (1, 1))
    used = jnp.sum((cnt > 0).astype(jnp.int32))
    ncode_ref[...] = jnp.reshape(used, (1, 1))
    ratio_ref[...] = jnp.reshape(used.astype(jnp.float32) / N_CODES, (1, 1))


def _stage3(x, emb_flat, counts2):
    return pl.pallas_call(
        _final_body,
        out_shape=(
            jax.ShapeDtypeStruct((N_TOKENS, D), jnp.float32),
            jax.ShapeDtypeStruct((1, 1), jnp.float32),
            jax.ShapeDtypeStruct((1, 1), jnp.float32),
            jax.ShapeDtypeStruct((1, 1), jnp.int32),
            jax.ShapeDtypeStruct((1, 1), jnp.float32),
        ),
    )(x, emb_flat, counts2)


def kernel(z, embeddings):
    b, c, h, w = z.shape
    x = jnp.moveaxis(z, 1, -1).reshape(-1, c)
    idx_flat = _stage1(x, embeddings)
    emb_flat, counts2 = _make_sc_gather()(embeddings, idx_flat)
    st_flat, loss, perp, ncodes, ratio = _stage3(x, emb_flat, counts2)
    encoding_indices = idx_flat.reshape(b, h, w)
    embeddings_st = jnp.moveaxis(st_flat.reshape(b, h, w, c), -1, 1)
    return (
        embeddings_st,
        encoding_indices,
        loss.reshape(()),
        perp.reshape(()),
        ncodes.reshape(()),
        ratio.reshape(()),
    )
